# Initial kernel scaffold; baseline (speedup 1.0000x reference)
import jax
import jax.numpy as jnp
from jax.experimental import pallas as pl


def kernel(features, edge_index, key_idx, e2_src, e2_dst, emb_table, W0, b0, ln_g0, ln_b0, W1, b1, ln_g1, ln_b1, W_sage, b_sage, ln2_g, ln2_b, W_info, b_info, aug_data, Wp1, bp1, Wp2, bp2):
    raise NotImplementedError("write your pallas kernel here")


# speedup vs baseline: 1.0246x; 1.0246x over previous
import functools

import jax
import jax.numpy as jnp
from jax.experimental import pallas as pl
from jax.experimental.pallas import tpu as pltpu

RANK = 128
B = 10000
N = B * 9
E = 320000
M = 5 ** 6 + 1


def _predictor_body(ge_ref, wa_ref, wb_ref, bp1_ref, info_ref, wp2_ref, bp2_ref, out_ref):
    ge = ge_ref[:, 0, :]
    c = jnp.dot(info_ref[...], wb_ref[...], preferred_element_type=jnp.float32) + bp1_ref[...]
    h = jnp.dot(ge, wa_ref[...], preferred_element_type=jnp.float32) + c
    h = jax.nn.relu(h)
    out_ref[...] = jnp.dot(h, wp2_ref[...], preferred_element_type=jnp.float32) + bp2_ref[...]


def _predictor(graph_embeds, info_vec, Wp1, bp1, Wp2, bp2):
    bn = 2000
    ge3 = graph_embeds.reshape(B, 1, RANK)
    wa = Wp1[:RANK]
    wb = Wp1[RANK:]
    return pl.pallas_call(
        _predictor_body,
        grid=(B // bn,),
        in_specs=[
            pl.BlockSpec((bn, 1, RANK), lambda i: (i, 0, 0)),
            pl.BlockSpec((RANK, RANK), lambda i: (0, 0)),
            pl.BlockSpec((RANK, RANK), lambda i: (0, 0)),
            pl.BlockSpec((RANK,), lambda i: (0,)),
            pl.BlockSpec((1, RANK), lambda i: (0, 0)),
            pl.BlockSpec((RANK, 1), lambda i: (0, 0)),
            pl.BlockSpec((1,), lambda i: (0,)),
        ],
        out_specs=pl.BlockSpec((bn, 1), lambda i: (i, 0)),
        out_shape=jax.ShapeDtypeStruct((B, 1), jnp.float32),
    )(ge3, wa, wb, bp1, info_vec.reshape(1, RANK), Wp2, bp2)


def _sage_dense(agg, feats, deg, W, b, g, bb):
    h = (agg + feats) / (deg[:, None] + 1.0)
    h = h @ W + b
    m = jnp.mean(h, axis=-1, keepdims=True)
    v = jnp.var(h, axis=-1, keepdims=True)
    h = (h - m) / jnp.sqrt(v + 1e-5) * g + bb
    return jax.nn.relu(h)


def kernel(features, edge_index, key_idx, e2_src, e2_dst, emb_table, W0, b0, ln_g0, ln_b0, W1, b1, ln_g1, ln_b1, W_sage, b_sage, ln2_g, ln2_b, W_info, b_info, aug_data, Wp1, bp1, Wp2, bp2):
    del e2_src, e2_dst
    feats = emb_table[features.reshape(-1)]
    src, dst = edge_index[0], edge_index[1]
    deg = jnp.zeros((N,), jnp.float32).at[dst].add(1.0)
    for (W, b, g, bb) in ((W0, b0, ln_g0, ln_b0), (W1, b1, ln_g1, ln_b1)):
        agg = jnp.zeros((N, RANK), jnp.float32).at[dst].add(feats[src])
        feats = _sage_dense(agg, feats, deg, W, b, g, bb)
    graph_embeds = feats.reshape(B, 9, RANK)[:, 0, :]
    iota = jnp.arange(B, dtype=jnp.int32)
    occ = jnp.full((M - 1,), -1, jnp.int32).at[key_idx].max(iota)
    mask = (occ[key_idx] == iota).astype(jnp.float32)
    S = jnp.sum(graph_embeds * mask[:, None], axis=0)
    info = (aug_data @ W_info + b_info)[0]
    hM = (S + info) / float(M)
    t = hM @ W_sage + b_sage
    m = jnp.mean(t)
    v = jnp.var(t)
    t = (t - m) / jnp.sqrt(v + 1e-5) * ln2_g + ln2_b
    info_vec = jax.nn.relu(t)
    return _predictor(graph_embeds, info_vec, Wp1, bp1, Wp2, bp2)


# speedup vs baseline: 2.0790x; 2.0290x over previous
import functools

import jax
import jax.numpy as jnp
from jax import lax
from jax.experimental import pallas as pl
from jax.experimental.pallas import tpu as pltpu
from jax.experimental.pallas import tpu_sc as plsc

RANK = 128
B = 10000
N = B * 9
E = 320000
M = 5 ** 6 + 1

NP = 90112
STRIPE = NP // 16
EP = 327680
K = 128
EPW = EP // K
TW = EPW // 16
HW = TW // 2
NBUF = 4
DW = EPW // 32

_MESH = plsc.VectorSubcoreMesh(
    core_axis_name="c", subcore_axis_name="s", num_cores=2, num_subcores=16)


def _fill(buf, value):
    v = jnp.full((16,), value, jnp.float32)
    for i in range(128):
        buf[i] = v


@functools.partial(
    pl.kernel,
    out_type=jax.ShapeDtypeStruct((2 * NP, 16), jnp.float32),
    mesh=_MESH,
    compiler_params=pltpu.CompilerParams(use_tc_tiling_on_sc=False),
    scratch_types=[
        pltpu.VMEM((DW, K), jnp.int32),
        pltpu.VMEM((128, 16), jnp.float32),
        pltpu.VMEM_SHARED((NP, 16), jnp.float32),
    ],
)
def _sc_deg(dst2d, out, dstbuf, ones, degbuf):
    c = lax.axis_index("c")
    s = lax.axis_index("s")
    pltpu.sync_copy(dst2d.at[pl.ds((c * 16 + s) * DW, DW)], dstbuf)
    _fill(ones, 0.0)
    def zr(i, carry):
        pltpu.sync_copy(ones, degbuf.at[pl.ds(s * STRIPE + i * 128, 128)])
        return carry
    lax.fori_loop(0, STRIPE // 128, zr, 0)
    _fill(ones, 1.0)
    plsc.subcore_barrier()
    def w_body(w, carry):
        pltpu.sync_copy(ones, degbuf.at[dstbuf.at[w]], add=True)
        return carry
    lax.fori_loop(0, DW, w_body, 0)
    plsc.subcore_barrier()
    pltpu.sync_copy(degbuf.at[pl.ds(s * STRIPE, STRIPE)],
                    out.at[pl.ds(c * NP + s * STRIPE, STRIPE)])


@functools.partial(
    pl.kernel,
    out_type=jax.ShapeDtypeStruct((8 * NP, 16), jnp.float32),
    mesh=_MESH,
    compiler_params=pltpu.CompilerParams(use_tc_tiling_on_sc=False),
    scratch_types=[
        pltpu.VMEM((HW * K,), jnp.int32),
        pltpu.VMEM((HW, K), jnp.int32),
        pltpu.VMEM((NBUF, K, 16), jnp.float32),
        pltpu.VMEM((128, 16), jnp.float32),
        pltpu.VMEM_SHARED((NP, 16), jnp.float32),
        pltpu.SemaphoreType.DMA((NBUF,)),
    ],
)
def _sc_agg(idx8, dst2d, feats, out, i8buf, dstbuf, rows, zbuf, aggbuf, sems):
    c = lax.axis_index("c")
    s = lax.axis_index("s")
    _fill(zbuf, 0.0)
    for g_loc in range(4):
        g = c * 4 + g_loc
        def zr(i, carry):
            pltpu.sync_copy(zbuf, aggbuf.at[pl.ds(s * STRIPE + i * 128, 128)])
            return carry
        lax.fori_loop(0, STRIPE // 128, zr, 0)
        plsc.subcore_barrier()
        for half in range(2):
            base_w = s * TW + half * HW
            pltpu.sync_copy(idx8.at[pl.ds(base_w * K, HW * K)], i8buf)
            pltpu.sync_copy(dst2d.at[pl.ds(base_w, HW)], dstbuf)
            def addg(j, carry):
                i8buf[pl.ds(j * 16, 16)] = i8buf[pl.ds(j * 16, 16)] + g
                return carry
            lax.fori_loop(0, HW * 8, addg, 0)
            for b in range(NBUF):
                pltpu.async_copy(feats.at[i8buf.at[pl.ds(b * K, K)]],
                                 rows.at[b], sems.at[b])
            def outer(wo, carry):
                for b in range(NBUF):
                    w = wo * NBUF + b
                    pltpu.make_async_copy(feats.at[pl.ds(0, K)],
                                          rows.at[b], sems.at[b]).wait()
                    pltpu.sync_copy(rows.at[b], aggbuf.at[dstbuf.at[w]],
                                    add=True)
                    @pl.when(wo < HW // NBUF - 1)
                    def _fire():
                        nxt = w + NBUF
                        pltpu.async_copy(feats.at[i8buf.at[pl.ds(nxt * K, K)]],
                                         rows.at[b], sems.at[b])
                return carry
            lax.fori_loop(0, HW // NBUF, outer, 0)
        plsc.subcore_barrier()
        off = pl.multiple_of(g * NP + s * STRIPE, 512)
        pltpu.sync_copy(aggbuf.at[pl.ds(s * STRIPE, STRIPE)],
                        out.at[pl.ds(off, STRIPE)])
        plsc.subcore_barrier()


def _embed_body(f_ref, emb_ref, out_ref):
    f = f_ref[...]
    onehot = (f == lax.broadcasted_iota(jnp.int32, (f.shape[0], 8), 1))
    out_ref[...] = jnp.dot(onehot.astype(jnp.float32), emb_ref[...],
                           preferred_element_type=jnp.float32,
                           precision=lax.Precision.HIGHEST)


def _embed(features_pad, emb8):
    bn = 1024
    return pl.pallas_call(
        _embed_body,
        grid=(NP // bn,),
        in_specs=[
            pl.BlockSpec((bn, 1), lambda i: (i, 0)),
            pl.BlockSpec((8, RANK), lambda i: (0, 0)),
        ],
        out_specs=pl.BlockSpec((bn, RANK), lambda i: (i, 0)),
        out_shape=jax.ShapeDtypeStruct((NP, RANK), jnp.float32),
    )(features_pad, emb8)


def _dense_body(a0, a1, a2, a3, a4, a5, a6, a7, feats_ref, d0_ref, d1_ref,
                w_ref, b_ref, g_ref, bb_ref, out_ref):
    deg = d0_ref[:, 0:1] + d1_ref[:, 0:1]
    agg = jnp.concatenate(
        [a[...] for a in (a0, a1, a2, a3, a4, a5, a6, a7)], axis=-1)
    h = (agg + feats_ref[...]) / (deg + 1.0)
    z = jnp.dot(h, w_ref[...], preferred_element_type=jnp.float32) + b_ref[...]
    m = jnp.mean(z, axis=-1, keepdims=True)
    v = jnp.mean((z - m) ** 2, axis=-1, keepdims=True)
    z = (z - m) / jnp.sqrt(v + 1e-5) * g_ref[...] + bb_ref[...]
    out_ref[...] = jax.nn.relu(z)


def _dense(agg8, feats, degp, W, b, g, bb):
    bn = 1024
    nblk = NP // bn
    agg_specs = [
        pl.BlockSpec((bn, 16), functools.partial(lambda gi, i: (gi * nblk + i, 0), gi))
        for gi in range(8)
    ]
    return pl.pallas_call(
        _dense_body,
        grid=(nblk,),
        in_specs=agg_specs + [
            pl.BlockSpec((bn, RANK), lambda i: (i, 0)),
            pl.BlockSpec((bn, 16), lambda i: (i, 0)),
            pl.BlockSpec((bn, 16), lambda i, _n=nblk: (i + _n, 0)),
            pl.BlockSpec((RANK, RANK), lambda i: (0, 0)),
            pl.BlockSpec((RANK,), lambda i: (0,)),
            pl.BlockSpec((RANK,), lambda i: (0,)),
            pl.BlockSpec((RANK,), lambda i: (0,)),
        ],
        out_specs=pl.BlockSpec((bn, RANK), lambda i: (i, 0)),
        out_shape=jax.ShapeDtypeStruct((NP, RANK), jnp.float32),
    )(*([agg8] * 8), feats, degp, degp, W, b, g, bb)


def _reduce_body(feats_ref, mask_ref, out_ref):
    i = pl.program_id(0)
    blk = feats_ref[...]
    ge = blk.reshape(blk.shape[0] // 9, 9, RANK)[:, 0, :]
    part = jnp.sum(ge * mask_ref[...], axis=0, keepdims=True)
    @pl.when(i == 0)
    def _init():
        out_ref[...] = jnp.zeros_like(out_ref)
    out_ref[...] += part


def _reduce(feats, mask):
    bn = 400
    return pl.pallas_call(
        _reduce_body,
        grid=(B // bn,),
        in_specs=[
            pl.BlockSpec((bn * 9, RANK), lambda i: (i, 0)),
            pl.BlockSpec((bn, 1), lambda i: (i, 0)),
        ],
        out_specs=pl.BlockSpec((1, RANK), lambda i: (0, 0)),
        out_shape=jax.ShapeDtypeStruct((1, RANK), jnp.float32),
    )(feats, mask)


def _info_body(s_ref, aug_ref, wi_ref, bi_ref, ws_ref, bs_ref, g_ref, bb_ref, out_ref):
    info = jnp.dot(aug_ref[...], wi_ref[...],
                   preferred_element_type=jnp.float32) + bi_ref[...]
    hM = (s_ref[...] + info) / float(M)
    t = jnp.dot(hM, ws_ref[...], preferred_element_type=jnp.float32) + bs_ref[...]
    m = jnp.mean(t, axis=-1, keepdims=True)
    v = jnp.mean((t - m) ** 2, axis=-1, keepdims=True)
    t = (t - m) / jnp.sqrt(v + 1e-5) * g_ref[...] + bb_ref[...]
    out_ref[...] = jax.nn.relu(t)


def _info(S, aug_data, W_info, b_info, W_sage, b_sage, ln2_g, ln2_b):
    return pl.pallas_call(
        _info_body,
        out_shape=jax.ShapeDtypeStruct((1, RANK), jnp.float32),
    )(S, aug_data, W_info, b_info.reshape(1, RANK), W_sage,
      b_sage.reshape(1, RANK), ln2_g.reshape(1, RANK), ln2_b.reshape(1, RANK))


def _predictor_body(feats_ref, wa_ref, wb_ref, bp1_ref, info_ref, wp2_ref, bp2_ref, out_ref):
    blk = feats_ref[...]
    ge = blk.reshape(blk.shape[0] // 9, 9, RANK)[:, 0, :]
    c = jnp.dot(info_ref[...], wb_ref[...], preferred_element_type=jnp.float32) + bp1_ref[...]
    h = jnp.dot(ge, wa_ref[...], preferred_element_type=jnp.float32) + c
    h = jax.nn.relu(h)
    out_ref[...] = jnp.dot(h, wp2_ref[...], preferred_element_type=jnp.float32) + bp2_ref[...]


def _predictor(feats, info_vec, Wp1, bp1, Wp2, bp2):
    bn = 400
    return pl.pallas_call(
        _predictor_body,
        grid=(B // bn,),
        in_specs=[
            pl.BlockSpec((bn * 9, RANK), lambda i: (i, 0)),
            pl.BlockSpec((RANK, RANK), lambda i: (0, 0)),
            pl.BlockSpec((RANK, RANK), lambda i: (0, 0)),
            pl.BlockSpec((RANK,), lambda i: (0,)),
            pl.BlockSpec((1, RANK), lambda i: (0, 0)),
            pl.BlockSpec((RANK, 1), lambda i: (0, 0)),
            pl.BlockSpec((1,), lambda i: (0,)),
        ],
        out_specs=pl.BlockSpec((bn, 1), lambda i: (i, 0)),
        out_shape=jax.ShapeDtypeStruct((B, 1), jnp.float32),
    )(feats, Wp1[:RANK], Wp1[RANK:], bp1, info_vec, Wp2, bp2)


def kernel(features, edge_index, key_idx, e2_src, e2_dst, emb_table, W0, b0, ln_g0, ln_b0, W1, b1, ln_g1, ln_b1, W_sage, b_sage, ln2_g, ln2_b, W_info, b_info, aug_data, Wp1, bp1, Wp2, bp2):
    del e2_src, e2_dst
    src, dst = edge_index[0], edge_index[1]
    pad = EP - E
    idx8 = jnp.concatenate([src * 8, jnp.zeros((pad,), jnp.int32)])
    dst2d = jnp.concatenate([dst, jnp.full((pad,), N, jnp.int32)]).reshape(EPW, K)
    features_pad = jnp.concatenate(
        [features.reshape(-1), jnp.zeros((NP - N,), jnp.int32)]).reshape(NP, 1)
    emb8 = jnp.concatenate([emb_table, jnp.zeros((2, RANK), jnp.float32)])

    degp = _sc_deg(dst2d)
    feats = _embed(features_pad, emb8)
    for (W, b, g, bb) in ((W0, b0, ln_g0, ln_b0), (W1, b1, ln_g1, ln_b1)):
        agg8 = _sc_agg(idx8, dst2d, feats.reshape(NP * 8, 16))
        feats = _dense(agg8, feats, degp, W, b, g, bb)

    iota = jnp.arange(B, dtype=jnp.int32)
    occ = jnp.full((M - 1,), -1, jnp.int32).at[key_idx].max(iota)
    mask = (occ[key_idx] == iota).astype(jnp.float32).reshape(B, 1)
    S = _reduce(feats, mask)
    info_vec = _info(S, aug_data, W_info, b_info, W_sage, b_sage, ln2_g, ln2_b)
    return _predictor(feats, info_vec, Wp1, bp1, Wp2, bp2)


# speedup vs baseline: 2.1010x; 1.0106x over previous
import functools

import jax
import jax.numpy as jnp
from jax import lax
from jax.experimental import pallas as pl
from jax.experimental.pallas import tpu as pltpu
from jax.experimental.pallas import tpu_sc as plsc

RANK = 128
B = 10000
N = B * 9
E = 320000
M = 5 ** 6 + 1

NP = 90112
STRIPE = NP // 16
EP = 327680
K = 128
EPW = EP // K
TW = EPW // 16
HW = TW // 2
RB = 8
LEAD = 4
DW = EPW // 32

_MESH = plsc.VectorSubcoreMesh(
    core_axis_name="c", subcore_axis_name="s", num_cores=2, num_subcores=16)


def _fill(buf, value):
    v = jnp.full((16,), value, jnp.float32)
    for i in range(128):
        buf[i] = v


@functools.partial(
    pl.kernel,
    out_type=jax.ShapeDtypeStruct((2 * NP, 16), jnp.float32),
    mesh=_MESH,
    compiler_params=pltpu.CompilerParams(use_tc_tiling_on_sc=False),
    scratch_types=[
        pltpu.VMEM((DW, K), jnp.int32),
        pltpu.VMEM((128, 16), jnp.float32),
        pltpu.VMEM_SHARED((NP, 16), jnp.float32),
    ],
)
def _sc_deg(dst2d, out, dstbuf, ones, degbuf):
    c = lax.axis_index("c")
    s = lax.axis_index("s")
    pltpu.sync_copy(dst2d.at[pl.ds((c * 16 + s) * DW, DW)], dstbuf)
    _fill(ones, 0.0)
    def zr(i, carry):
        pltpu.sync_copy(ones, degbuf.at[pl.ds(s * STRIPE + i * 128, 128)])
        return carry
    lax.fori_loop(0, STRIPE // 128, zr, 0)
    _fill(ones, 1.0)
    plsc.subcore_barrier()
    def w_body(w, carry):
        pltpu.sync_copy(ones, degbuf.at[dstbuf.at[w]], add=True)
        return carry
    lax.fori_loop(0, DW, w_body, 0)
    plsc.subcore_barrier()
    pltpu.sync_copy(degbuf.at[pl.ds(s * STRIPE, STRIPE)],
                    out.at[pl.ds(c * NP + s * STRIPE, STRIPE)])


@functools.partial(
    pl.kernel,
    out_type=jax.ShapeDtypeStruct((8 * NP, 16), jnp.float32),
    mesh=_MESH,
    compiler_params=pltpu.CompilerParams(use_tc_tiling_on_sc=False),
    scratch_types=[
        pltpu.VMEM((HW * K,), jnp.int32),
        pltpu.VMEM((HW, K), jnp.int32),
        pltpu.VMEM((RB, K, 16), jnp.float32),
        pltpu.VMEM((128, 16), jnp.float32),
        pltpu.VMEM_SHARED((NP, 16), jnp.float32),
        pltpu.SemaphoreType.DMA((RB,)),
        pltpu.SemaphoreType.DMA((RB,)),
    ],
)
def _sc_agg(idx8, dst2d, feats, out, i8buf, dstbuf, rows, zbuf, aggbuf, sg, ss):
    c = lax.axis_index("c")
    s = lax.axis_index("s")
    _fill(zbuf, 0.0)

    def fire_gather(b, w):
        pltpu.async_copy(feats.at[i8buf.at[pl.ds(w * K, K)]], rows.at[b],
                         sg.at[b])

    def wait_gather(b):
        pltpu.make_async_copy(feats.at[pl.ds(0, K)], rows.at[b],
                              sg.at[b]).wait()

    def fire_scatter(b, w):
        pltpu.async_copy(rows.at[b], aggbuf.at[dstbuf.at[w]], ss.at[b],
                         add=True)

    def wait_scatter(b, w):
        pltpu.make_async_copy(rows.at[b], aggbuf.at[dstbuf.at[w]],
                              ss.at[b]).wait()

    for g_loc in range(4):
        g = c * 4 + g_loc
        def zr(i, carry):
            pltpu.sync_copy(zbuf, aggbuf.at[pl.ds(s * STRIPE + i * 128, 128)])
            return carry
        lax.fori_loop(0, STRIPE // 128, zr, 0)
        plsc.subcore_barrier()
        for half in range(2):
            base_w = s * TW + half * HW
            pltpu.sync_copy(idx8.at[pl.ds(base_w * K, HW * K)], i8buf)
            pltpu.sync_copy(dst2d.at[pl.ds(base_w, HW)], dstbuf)
            def addg(j, carry):
                i8buf[pl.ds(j * 16, 16)] = i8buf[pl.ds(j * 16, 16)] + g
                return carry
            lax.fori_loop(0, HW * 8, addg, 0)
            for b in range(LEAD):
                fire_gather(b, b)
            def outer(wo, carry):
                for b in range(RB):
                    w = wo * RB + b
                    wait_gather(b)
                    fire_scatter(b, w)
                    bn_ = (b + LEAD) % RB
                    wn = w + LEAD
                    @pl.when(wn >= RB)
                    def _drain():
                        wait_scatter(bn_, wn - RB)
                    @pl.when(wn < HW)
                    def _refire():
                        fire_gather(bn_, wn)
                return carry
            lax.fori_loop(0, HW // RB, outer, 0)
            for b in range(RB - LEAD, RB):
                wait_scatter(b, HW - RB + b)
        plsc.subcore_barrier()
        off = pl.multiple_of(g * NP + s * STRIPE, 512)
        pltpu.sync_copy(aggbuf.at[pl.ds(s * STRIPE, STRIPE)],
                        out.at[pl.ds(off, STRIPE)])
        plsc.subcore_barrier()


def _embed_body(f_ref, emb_ref, out_ref):
    f = f_ref[...]
    onehot = (f == lax.broadcasted_iota(jnp.int32, (f.shape[0], 8), 1))
    out_ref[...] = jnp.dot(onehot.astype(jnp.float32), emb_ref[...],
                           preferred_element_type=jnp.float32,
                           precision=lax.Precision.HIGHEST)


def _embed(features_pad, emb8):
    bn = 1024
    return pl.pallas_call(
        _embed_body,
        grid=(NP // bn,),
        in_specs=[
            pl.BlockSpec((bn, 1), lambda i: (i, 0)),
            pl.BlockSpec((8, RANK), lambda i: (0, 0)),
        ],
        out_specs=pl.BlockSpec((bn, RANK), lambda i: (i, 0)),
        out_shape=jax.ShapeDtypeStruct((NP, RANK), jnp.float32),
    )(features_pad, emb8)


def _dense_body(a0, a1, a2, a3, a4, a5, a6, a7, feats_ref, d0_ref, d1_ref,
                w_ref, b_ref, g_ref, bb_ref, out_ref):
    deg = d0_ref[:, 0:1] + d1_ref[:, 0:1]
    agg = jnp.concatenate(
        [a[...] for a in (a0, a1, a2, a3, a4, a5, a6, a7)], axis=-1)
    h = (agg + feats_ref[...]) / (deg + 1.0)
    z = jnp.dot(h, w_ref[...], preferred_element_type=jnp.float32) + b_ref[...]
    m = jnp.mean(z, axis=-1, keepdims=True)
    v = jnp.mean((z - m) ** 2, axis=-1, keepdims=True)
    z = (z - m) / jnp.sqrt(v + 1e-5) * g_ref[...] + bb_ref[...]
    out_ref[...] = jax.nn.relu(z)


def _dense(agg8, feats, degp, W, b, g, bb):
    bn = 1024
    nblk = NP // bn
    agg_specs = [
        pl.BlockSpec((bn, 16), functools.partial(lambda gi, i: (gi * nblk + i, 0), gi))
        for gi in range(8)
    ]
    return pl.pallas_call(
        _dense_body,
        grid=(nblk,),
        in_specs=agg_specs + [
            pl.BlockSpec((bn, RANK), lambda i: (i, 0)),
            pl.BlockSpec((bn, 16), lambda i: (i, 0)),
            pl.BlockSpec((bn, 16), lambda i, _n=nblk: (i + _n, 0)),
            pl.BlockSpec((RANK, RANK), lambda i: (0, 0)),
            pl.BlockSpec((RANK,), lambda i: (0,)),
            pl.BlockSpec((RANK,), lambda i: (0,)),
            pl.BlockSpec((RANK,), lambda i: (0,)),
        ],
        out_specs=pl.BlockSpec((bn, RANK), lambda i: (i, 0)),
        out_shape=jax.ShapeDtypeStruct((NP, RANK), jnp.float32),
    )(*([agg8] * 8), feats, degp, degp, W, b, g, bb)


def _reduce_body(feats_ref, mask_ref, out_ref):
    i = pl.program_id(0)
    blk = feats_ref[...]
    ge = blk.reshape(blk.shape[0] // 9, 9, RANK)[:, 0, :]
    part = jnp.sum(ge * mask_ref[...], axis=0, keepdims=True)
    @pl.when(i == 0)
    def _init():
        out_ref[...] = jnp.zeros_like(out_ref)
    out_ref[...] += part


def _reduce(feats, mask):
    bn = 400
    return pl.pallas_call(
        _reduce_body,
        grid=(B // bn,),
        in_specs=[
            pl.BlockSpec((bn * 9, RANK), lambda i: (i, 0)),
            pl.BlockSpec((bn, 1), lambda i: (i, 0)),
        ],
        out_specs=pl.BlockSpec((1, RANK), lambda i: (0, 0)),
        out_shape=jax.ShapeDtypeStruct((1, RANK), jnp.float32),
    )(feats, mask)


def _info_body(s_ref, aug_ref, wi_ref, bi_ref, ws_ref, bs_ref, g_ref, bb_ref, out_ref):
    info = jnp.dot(aug_ref[...], wi_ref[...],
                   preferred_element_type=jnp.float32) + bi_ref[...]
    hM = (s_ref[...] + info) / float(M)
    t = jnp.dot(hM, ws_ref[...], preferred_element_type=jnp.float32) + bs_ref[...]
    m = jnp.mean(t, axis=-1, keepdims=True)
    v = jnp.mean((t - m) ** 2, axis=-1, keepdims=True)
    t = (t - m) / jnp.sqrt(v + 1e-5) * g_ref[...] + bb_ref[...]
    out_ref[...] = jax.nn.relu(t)


def _info(S, aug_data, W_info, b_info, W_sage, b_sage, ln2_g, ln2_b):
    return pl.pallas_call(
        _info_body,
        out_shape=jax.ShapeDtypeStruct((1, RANK), jnp.float32),
    )(S, aug_data, W_info, b_info.reshape(1, RANK), W_sage,
      b_sage.reshape(1, RANK), ln2_g.reshape(1, RANK), ln2_b.reshape(1, RANK))


def _predictor_body(feats_ref, wa_ref, wb_ref, bp1_ref, info_ref, wp2_ref, bp2_ref, out_ref):
    blk = feats_ref[...]
    ge = blk.reshape(blk.shape[0] // 9, 9, RANK)[:, 0, :]
    c = jnp.dot(info_ref[...], wb_ref[...], preferred_element_type=jnp.float32) + bp1_ref[...]
    h = jnp.dot(ge, wa_ref[...], preferred_element_type=jnp.float32) + c
    h = jax.nn.relu(h)
    out_ref[...] = jnp.dot(h, wp2_ref[...], preferred_element_type=jnp.float32) + bp2_ref[...]


def _predictor(feats, info_vec, Wp1, bp1, Wp2, bp2):
    bn = 400
    return pl.pallas_call(
        _predictor_body,
        grid=(B // bn,),
        in_specs=[
            pl.BlockSpec((bn * 9, RANK), lambda i: (i, 0)),
            pl.BlockSpec((RANK, RANK), lambda i: (0, 0)),
            pl.BlockSpec((RANK, RANK), lambda i: (0, 0)),
            pl.BlockSpec((RANK,), lambda i: (0,)),
            pl.BlockSpec((1, RANK), lambda i: (0, 0)),
            pl.BlockSpec((RANK, 1), lambda i: (0, 0)),
            pl.BlockSpec((1,), lambda i: (0,)),
        ],
        out_specs=pl.BlockSpec((bn, 1), lambda i: (i, 0)),
        out_shape=jax.ShapeDtypeStruct((B, 1), jnp.float32),
    )(feats, Wp1[:RANK], Wp1[RANK:], bp1, info_vec, Wp2, bp2)


def kernel(features, edge_index, key_idx, e2_src, e2_dst, emb_table, W0, b0, ln_g0, ln_b0, W1, b1, ln_g1, ln_b1, W_sage, b_sage, ln2_g, ln2_b, W_info, b_info, aug_data, Wp1, bp1, Wp2, bp2):
    del e2_src, e2_dst
    src, dst = edge_index[0], edge_index[1]
    pad = EP - E
    idx8 = jnp.concatenate([src * 8, jnp.zeros((pad,), jnp.int32)])
    dst2d = jnp.concatenate([dst, jnp.full((pad,), N, jnp.int32)]).reshape(EPW, K)
    features_pad = jnp.concatenate(
        [features.reshape(-1), jnp.zeros((NP - N,), jnp.int32)]).reshape(NP, 1)
    emb8 = jnp.concatenate([emb_table, jnp.zeros((2, RANK), jnp.float32)])

    degp = _sc_deg(dst2d)
    feats = _embed(features_pad, emb8)
    for (W, b, g, bb) in ((W0, b0, ln_g0, ln_b0), (W1, b1, ln_g1, ln_b1)):
        agg8 = _sc_agg(idx8, dst2d, feats.reshape(NP * 8, 16))
        feats = _dense(agg8, feats, degp, W, b, g, bb)

    iota = jnp.arange(B, dtype=jnp.int32)
    occ = jnp.full((M - 1,), -1, jnp.int32).at[key_idx].max(iota)
    mask = (occ[key_idx] == iota).astype(jnp.float32).reshape(B, 1)
    S = _reduce(feats, mask)
    info_vec = _info(S, aug_data, W_info, b_info, W_sage, b_sage, ln2_g, ln2_b)
    return _predictor(feats, info_vec, Wp1, bp1, Wp2, bp2)
